# Initial kernel scaffold; baseline (speedup 1.0000x reference)
import jax
import jax.numpy as jnp
from jax.experimental import pallas as pl


def kernel(x, adj, W1, a1):
    raise NotImplementedError("write your pallas kernel here")


# speedup vs baseline: 1.9453x; 1.9453x over previous
import functools

import jax
import jax.numpy as jnp
from jax.experimental import pallas as pl
from jax.experimental.pallas import tpu as pltpu

_NEG = -9e15

_R0 = 1000
_R = 200


def _proj_kernel(h_ref, w_ref, asrc_ref, adst_ref, wh_ref, s1_ref, s2_ref):
    wh = jnp.dot(h_ref[...], w_ref[...], preferred_element_type=jnp.float32)
    wh_ref[...] = wh
    s1_ref[...] = jnp.dot(wh, asrc_ref[...], preferred_element_type=jnp.float32)
    s2_ref[...] = jnp.dot(wh, adst_ref[...], preferred_element_type=jnp.float32)


def _proj(h, W, a_src, a_dst):
    n, d_in = h.shape
    d_out = W.shape[1]
    grid = (n // _R0,)
    return pl.pallas_call(
        _proj_kernel,
        grid=grid,
        in_specs=[
            pl.BlockSpec((_R0, d_in), lambda i: (i, 0)),
            pl.BlockSpec((d_in, d_out), lambda i: (0, 0)),
            pl.BlockSpec((d_out, 1), lambda i: (0, 0)),
            pl.BlockSpec((d_out, 1), lambda i: (0, 0)),
        ],
        out_specs=[
            pl.BlockSpec((_R0, d_out), lambda i: (i, 0)),
            pl.BlockSpec((_R0, 1), lambda i: (i, 0)),
            pl.BlockSpec((_R0, 1), lambda i: (i, 0)),
        ],
        out_shape=[
            jax.ShapeDtypeStruct((n, d_out), jnp.float32),
            jax.ShapeDtypeStruct((n, 1), jnp.float32),
            jax.ShapeDtypeStruct((n, 1), jnp.float32),
        ],
    )(h, W, a_src, a_dst)


def _att_rows(s1_ref, s2t_ref, maskb):
    e = s1_ref[...] + s2t_ref[...]
    e = jnp.where(e > 0.0, e, 0.2 * e)
    e = jnp.where(maskb, e, _NEG)
    m = jnp.max(e, axis=1, keepdims=True)
    p = jnp.exp(e - m)
    l = jnp.sum(p, axis=1, keepdims=True)
    return p, l


def _layer1_kernel(s1_ref, s2t_ref, adj_ref, wh_ref, out_ref, mask_ref):
    maskb = adj_ref[...] > 0
    mask_ref[...] = maskb.astype(jnp.int8)
    p, l = _att_rows(s1_ref, s2t_ref, maskb)
    h = jnp.dot(p, wh_ref[...], preferred_element_type=jnp.float32) / l
    out_ref[...] = jnp.where(h > 0.0, h, jnp.exp(h) - 1.0)


def _layer2_kernel(s1_ref, s2t_ref, mask_ref, wh_ref, out_ref, sum_ref, *,
                   inv_n):
    i = pl.program_id(0)

    @pl.when(i == 0)
    def _():
        sum_ref[...] = jnp.zeros(sum_ref.shape, jnp.float32)

    maskb = mask_ref[...].astype(jnp.int32) > 0
    p, l = _att_rows(s1_ref, s2t_ref, maskb)
    h = jnp.dot(p, wh_ref[...], preferred_element_type=jnp.float32) / l
    sum_ref[...] += jnp.sum(h, axis=0, keepdims=True)

    @pl.when(i == pl.num_programs(0) - 1)
    def _():
        out_ref[...] = sum_ref[...] * inv_n


def _layer1(s1, s2t, adj, wh):
    n = adj.shape[0]
    d = wh.shape[1]
    return pl.pallas_call(
        _layer1_kernel,
        grid=(n // _R,),
        in_specs=[
            pl.BlockSpec((_R, 1), lambda i: (i, 0)),
            pl.BlockSpec((1, n), lambda i: (0, 0)),
            pl.BlockSpec((_R, n), lambda i: (i, 0)),
            pl.BlockSpec((n, d), lambda i: (0, 0)),
        ],
        out_specs=[
            pl.BlockSpec((_R, d), lambda i: (i, 0)),
            pl.BlockSpec((_R, n), lambda i: (i, 0)),
        ],
        out_shape=[
            jax.ShapeDtypeStruct((n, d), jnp.float32),
            jax.ShapeDtypeStruct((n, n), jnp.int8),
        ],
        compiler_params=pltpu.CompilerParams(
            dimension_semantics=("arbitrary",)),
    )(s1, s2t, adj, wh)


def _layer2_pooled(s1, s2t, mask_i8, wh):
    n = mask_i8.shape[0]
    d = wh.shape[1]
    kern = functools.partial(_layer2_kernel, inv_n=1.0 / n)
    return pl.pallas_call(
        kern,
        grid=(n // _R,),
        in_specs=[
            pl.BlockSpec((_R, 1), lambda i: (i, 0)),
            pl.BlockSpec((1, n), lambda i: (0, 0)),
            pl.BlockSpec((_R, n), lambda i: (i, 0)),
            pl.BlockSpec((n, d), lambda i: (0, 0)),
        ],
        out_specs=pl.BlockSpec((1, d), lambda i: (0, 0)),
        out_shape=jax.ShapeDtypeStruct((1, d), jnp.float32),
        scratch_shapes=[pltpu.VMEM((1, d), jnp.float32)],
        compiler_params=pltpu.CompilerParams(
            dimension_semantics=("arbitrary",)),
    )(s1, s2t, mask_i8, wh)


def kernel(x, adj, W1, a1):
    d_out = W1.shape[1]
    a_src = a1[:d_out]
    a_dst = a1[d_out:]

    wh1, s1_1, s2_1 = _proj(x, W1, a_src, a_dst)
    h1, mask_i8 = _layer1(s1_1, s2_1.T, adj, wh1)

    wh2, s1_2, s2_2 = _proj(h1, W1, a_src, a_dst)
    pooled = _layer2_pooled(s1_2, s2_2.T, mask_i8, wh2)
    return pooled.reshape(d_out)


# speedup vs baseline: 3.2140x; 1.6522x over previous
import functools

import jax
import jax.numpy as jnp
from jax.experimental import pallas as pl
from jax.experimental.pallas import tpu as pltpu

_LOG2E = 1.4426950408889634

_R0 = 1000
_R = 200


def _proj_kernel(h_ref, w_ref, asrc_ref, adst_ref,
                 whx_ref, s1_ref, s2_ref, s2max_ref):
    i = pl.program_id(0)
    d = w_ref.shape[1]
    wh = jnp.dot(h_ref[...], w_ref[...], preferred_element_type=jnp.float32)
    whx_ref[:, :d] = wh
    ones_col = (jax.lax.broadcasted_iota(jnp.int32, (h_ref.shape[0], d), 1)
                == 0).astype(jnp.float32)
    whx_ref[:, d:] = ones_col
    s1_ref[...] = jnp.dot(wh, asrc_ref[...],
                          preferred_element_type=jnp.float32) * _LOG2E
    s2 = jnp.dot(wh, adst_ref[...],
                 preferred_element_type=jnp.float32) * _LOG2E
    s2_ref[...] = s2
    bmax = jnp.max(s2, axis=0, keepdims=True)

    @pl.when(i == 0)
    def _():
        s2max_ref[...] = bmax

    @pl.when(i > 0)
    def _():
        s2max_ref[...] = jnp.maximum(s2max_ref[...], bmax)


def _proj(h, W, a_src, a_dst):
    n, d_in = h.shape
    d = W.shape[1]
    grid = (n // _R0,)
    return pl.pallas_call(
        _proj_kernel,
        grid=grid,
        in_specs=[
            pl.BlockSpec((_R0, d_in), lambda i: (i, 0)),
            pl.BlockSpec((d_in, d), lambda i: (0, 0)),
            pl.BlockSpec((d, 1), lambda i: (0, 0)),
            pl.BlockSpec((d, 1), lambda i: (0, 0)),
        ],
        out_specs=[
            pl.BlockSpec((_R0, 2 * d), lambda i: (i, 0)),
            pl.BlockSpec((_R0, 1), lambda i: (i, 0)),
            pl.BlockSpec((_R0, 1), lambda i: (i, 0)),
            pl.BlockSpec((1, 1), lambda i: (0, 0)),
        ],
        out_shape=[
            jax.ShapeDtypeStruct((n, 2 * d), jnp.float32),
            jax.ShapeDtypeStruct((n, 1), jnp.float32),
            jax.ShapeDtypeStruct((n, 1), jnp.float32),
            jax.ShapeDtypeStruct((1, 1), jnp.float32),
        ],
        compiler_params=pltpu.CompilerParams(
            dimension_semantics=("arbitrary",)),
    )(h, W, a_src, a_dst)


def _att_block(s1_ref, s2t_ref, s2max_ref, maskf, whx_ref):
    d = whx_ref.shape[1] // 2
    x0 = s1_ref[...] + s2max_ref[...]
    m = jnp.maximum(x0, 0.2 * x0)
    x = (s1_ref[...] - m) + s2t_ref[...]
    x2 = (0.2 * s1_ref[...] - m) + 0.2 * s2t_ref[...]
    p = jnp.exp2(jnp.maximum(x, x2)) * maskf
    out = jnp.dot(p, whx_ref[...], preferred_element_type=jnp.float32)
    return out[:, :d], out[:, d:d + 1]


def _layer1_kernel(s1_ref, s2t_ref, s2max_ref, adj_ref, whx_ref,
                   out_ref, mask_ref):
    adj = adj_ref[...]
    mask_ref[...] = adj.astype(jnp.int8)
    acc, l = _att_block(s1_ref, s2t_ref, s2max_ref,
                        adj.astype(jnp.float32), whx_ref)
    h = acc * (1.0 / l)
    out_ref[...] = jnp.where(h > 0.0, h, jnp.exp(h) - 1.0)


def _layer2_kernel(s1_ref, s2t_ref, s2max_ref, mask_ref, whx_ref,
                   out_ref, sum_ref, *, inv_n):
    i = pl.program_id(0)

    @pl.when(i == 0)
    def _():
        sum_ref[...] = jnp.zeros(sum_ref.shape, jnp.float32)

    acc, l = _att_block(s1_ref, s2t_ref, s2max_ref,
                        mask_ref[...].astype(jnp.float32), whx_ref)
    h = acc * (1.0 / l)
    sum_ref[...] += jnp.sum(h, axis=0, keepdims=True)

    @pl.when(i == pl.num_programs(0) - 1)
    def _():
        out_ref[...] = sum_ref[...] * inv_n


def _layer1(s1, s2t, s2max, adj, whx):
    n = adj.shape[0]
    d = whx.shape[1] // 2
    return pl.pallas_call(
        _layer1_kernel,
        grid=(n // _R,),
        in_specs=[
            pl.BlockSpec((_R, 1), lambda i: (i, 0)),
            pl.BlockSpec((1, n), lambda i: (0, 0)),
            pl.BlockSpec((1, 1), lambda i: (0, 0)),
            pl.BlockSpec((_R, n), lambda i: (i, 0)),
            pl.BlockSpec((n, 2 * d), lambda i: (0, 0)),
        ],
        out_specs=[
            pl.BlockSpec((_R, d), lambda i: (i, 0)),
            pl.BlockSpec((_R, n), lambda i: (i, 0)),
        ],
        out_shape=[
            jax.ShapeDtypeStruct((n, d), jnp.float32),
            jax.ShapeDtypeStruct((n, n), jnp.int8),
        ],
        compiler_params=pltpu.CompilerParams(
            dimension_semantics=("arbitrary",)),
    )(s1, s2t, s2max, adj, whx)


def _layer2_pooled(s1, s2t, s2max, mask_i8, whx):
    n = mask_i8.shape[0]
    d = whx.shape[1] // 2
    kern = functools.partial(_layer2_kernel, inv_n=1.0 / n)
    return pl.pallas_call(
        kern,
        grid=(n // _R,),
        in_specs=[
            pl.BlockSpec((_R, 1), lambda i: (i, 0)),
            pl.BlockSpec((1, n), lambda i: (0, 0)),
            pl.BlockSpec((1, 1), lambda i: (0, 0)),
            pl.BlockSpec((_R, n), lambda i: (i, 0)),
            pl.BlockSpec((n, 2 * d), lambda i: (0, 0)),
        ],
        out_specs=pl.BlockSpec((1, d), lambda i: (0, 0)),
        out_shape=jax.ShapeDtypeStruct((1, d), jnp.float32),
        scratch_shapes=[pltpu.VMEM((1, d), jnp.float32)],
        compiler_params=pltpu.CompilerParams(
            dimension_semantics=("arbitrary",)),
    )(s1, s2t, s2max, mask_i8, whx)


def kernel(x, adj, W1, a1):
    d = W1.shape[1]
    a_src = a1[:d]
    a_dst = a1[d:]

    whx1, s1_1, s2_1, s2max1 = _proj(x, W1, a_src, a_dst)
    h1, mask_i8 = _layer1(s1_1, s2_1.T, s2max1, adj, whx1)

    whx2, s1_2, s2_2, s2max2 = _proj(h1, W1, a_src, a_dst)
    pooled = _layer2_pooled(s1_2, s2_2.T, s2max2, mask_i8, whx2)
    return pooled.reshape(d)
